# Initial kernel scaffold; baseline (speedup 1.0000x reference)
import jax
import jax.numpy as jnp
from jax.experimental import pallas as pl


def kernel(indices, table):
    raise NotImplementedError("write your pallas kernel here")


# speedup vs baseline: 4.6554x; 4.6554x over previous
import functools

import jax
import jax.numpy as jnp
from jax import lax
from jax.experimental import pallas as pl
from jax.experimental.pallas import tpu as pltpu
from jax.experimental.pallas import tpu_sc as plsc

BATCH = 4096
HIST = 50
D = 64

NC = 2
NS = 16
NW = NC * NS
N_TOTAL = BATCH * HIST
PER_W = N_TOTAL // NW
CHUNK = 128
N_CH = PER_W // CHUNK
K = 10
N_OUTER = N_CH // K

_mesh = plsc.VectorSubcoreMesh(core_axis_name="c", subcore_axis_name="s")


@functools.partial(
    pl.kernel,
    mesh=_mesh,
    out_type=jax.ShapeDtypeStruct((N_TOTAL, D), jnp.float32),
    scratch_types=[
        pltpu.VMEM((N_CH, CHUNK), jnp.int32),
        pltpu.VMEM((K * CHUNK, D), jnp.float32),
        pltpu.SemaphoreType.DMA,
    ],
    compiler_params=pltpu.CompilerParams(use_tc_tiling_on_sc=False),
)
def _emb_gather(table_hbm, idx_hbm, out_hbm, idx_v, rows_v, sem):
    wid = lax.axis_index("s") * NC + lax.axis_index("c")
    base = wid * PER_W
    pltpu.sync_copy(idx_hbm.at[wid], idx_v)

    def outer(it, carry):
        copies = [
            pltpu.async_copy(
                table_hbm.at[idx_v.at[it * K + j]],
                rows_v.at[pl.ds(j * CHUNK, CHUNK)],
                sem,
            )
            for j in range(K)
        ]
        for cp in copies:
            cp.wait()
        pltpu.sync_copy(
            rows_v, out_hbm.at[pl.ds(base + it * (K * CHUNK), K * CHUNK)]
        )
        return carry

    lax.fori_loop(0, N_OUTER, outer, 0)


def kernel(indices, table):
    idx = indices.astype(jnp.int32).reshape(NW, N_CH, CHUNK)
    out = _emb_gather(table, idx)
    return out.reshape(BATCH, HIST, D)
